# Initial kernel scaffold; baseline (speedup 1.0000x reference)
import functools

import jax
import jax.numpy as jnp
from jax import lax
from jax.experimental import pallas as pl
from jax.experimental.pallas import tpu as pltpu
from jax.experimental.pallas import tpu_sc as plsc

NN = 100000
EE = 1600000
NC, NS = 2, 16
NHALF = NN // 2
TRASH = NHALF
ACC_ROWS = NHALF + 8
R = EE // 128
RPT = -(-R // NS)
R_PAD = RPT * NS
E_PAD = R_PAD * 128
RPN = NHALF // NS
HALF_RPT = RPT // 2

_SC_MESH = plsc.VectorSubcoreMesh(core_axis_name="c", subcore_axis_name="s")
_SC_PARAMS = pltpu.CompilerParams(use_tc_tiling_on_sc=False)


def _zero_acc(acc, zbuf, s, width):
    for r in range(128):
        for k in range(width // 16):
            zbuf[r, pl.ds(k * 16, 16)] = jnp.zeros((16,), jnp.float32)
    nfull, rem = RPN // 128, RPN % 128
    for i in range(nfull):
        pltpu.sync_copy(zbuf, acc.at[pl.ds(s * RPN + i * 128, 128)])
    if rem:
        pltpu.sync_copy(zbuf.at[pl.ds(0, rem)],
                        acc.at[pl.ds(s * RPN + nfull * 128, rem)])

    @pl.when(s == 0)
    def _():
        pltpu.sync_copy(zbuf.at[pl.ds(0, 8)], acc.at[pl.ds(NHALF, 8)])


@functools.partial(
    pl.kernel,
    out_type=(
        jax.ShapeDtypeStruct((NC, R_PAD, 128), jnp.int32),
        jax.ShapeDtypeStruct((NN, 16), jnp.float32),
    ),
    mesh=_SC_MESH,
    scratch_types=[
        pltpu.VMEM_SHARED((ACC_ROWS, 16), jnp.float32),
        pltpu.VMEM((RPT, 128), jnp.int32),
        pltpu.VMEM((256, 16), jnp.float32),
        pltpu.VMEM((128, 16), jnp.float32),
    ],
    compiler_params=_SC_PARAMS,
)
def _sc_prep_pass0(dst_hbm, ea_hbm, dstl_hbm, m0_hbm, acc, idx, ebuf, zbuf):
    c = lax.axis_index("c")
    s = lax.axis_index("s")
    _zero_acc(acc, zbuf, s, 16)
    pltpu.sync_copy(dst_hbm.at[pl.ds(s * RPT, RPT)], idx)
    base = c * NHALF

    def clamp_row(j, _):
        for k in range(8):
            d16 = idx[j, pl.ds(k * 16, 16)]
            v = d16 - base
            ok = (v >= 0) & (v < NHALF)
            idx[j, pl.ds(k * 16, 16)] = jnp.where(ok, v, TRASH)
        return 0

    lax.fori_loop(0, RPT, clamp_row, 0)
    pltpu.sync_copy(idx, dstl_hbm.at[c, pl.ds(s * RPT, RPT)])
    plsc.subcore_barrier()

    def scatter_group(g, _):
        e0 = (s * RPT + g * 2) * 128
        pltpu.sync_copy(ea_hbm.at[pl.ds(e0, 256)], ebuf)
        for jj in range(2):
            pltpu.sync_copy(ebuf.at[pl.ds(jj * 128, 128)],
                            acc.at[idx.at[g * 2 + jj]], add=True)
        return 0

    lax.fori_loop(0, RPT // 2, scatter_group, 0)
    plsc.subcore_barrier()
    pltpu.sync_copy(acc.at[pl.ds(s * RPN, RPN)],
                    m0_hbm.at[pl.ds(base + s * RPN, RPN)])


def _make_sc_pass(width):
    @functools.partial(
        pl.kernel,
        out_type=jax.ShapeDtypeStruct((NN, width), jnp.float32),
        mesh=_SC_MESH,
        scratch_types=[
            pltpu.VMEM_SHARED((ACC_ROWS, width), jnp.float32),
            pltpu.VMEM((HALF_RPT, 128), jnp.int32),
            pltpu.VMEM((HALF_RPT, 128), jnp.int32),
            pltpu.VMEM((128, width), jnp.float32),
            pltpu.VMEM((128, width), jnp.float32),
            pltpu.SemaphoreType.DMA,
        ],
        compiler_params=_SC_PARAMS,
    )
    def sc_pass(src_hbm, dstl_hbm, x_hbm, m_hbm, acc, srcv, dstv, rows, zbuf, sem):
        c = lax.axis_index("c")
        s = lax.axis_index("s")
        _zero_acc(acc, zbuf, s, width)
        plsc.subcore_barrier()
        for half in range(2):
            r0 = s * RPT + half * HALF_RPT
            pltpu.sync_copy(src_hbm.at[pl.ds(r0, HALF_RPT)], srcv)
            pltpu.sync_copy(dstl_hbm.at[c, pl.ds(r0, HALF_RPT)], dstv)

            def body(j, _):
                pltpu.async_copy(x_hbm.at[srcv.at[j]], rows, sem).wait()
                pltpu.sync_copy(rows, acc.at[dstv.at[j]], add=True)
                return 0

            lax.fori_loop(0, HALF_RPT, body, 0)
        plsc.subcore_barrier()
        pltpu.sync_copy(acc.at[pl.ds(s * RPN, RPN)],
                        m_hbm.at[pl.ds(c * NHALF + s * RPN, RPN)])

    return sc_pass


_sc_pass32 = _make_sc_pass(32)
_sc_pass16 = _make_sc_pass(16)

BLK = 1000
GRID = NN // BLK


def _row_spec(w):
    return pl.BlockSpec((BLK, w), lambda i: (i, 0))


def _full_spec(shape):
    nd = len(shape)
    return pl.BlockSpec(shape, lambda i: (0,) * nd)


def _tc1_body(md_ref, ni_ref, w0_ref, b0_ref, c0_ref, c1_ref, c2_ref, r_ref):
    md = md_ref[...]
    r = 1.0 / jnp.maximum(md[:, 5:6], 1.0)
    m0 = md[:, 0:5] * r
    h = jax.nn.relu(jnp.dot(m0, w0_ref[...],
                            preferred_element_type=jnp.float32) + b0_ref[...])
    c0_ref[...] = h[:, :32]
    c1_ref[...] = h[:, 32:]
    c2_ref[...] = jnp.concatenate(
        [ni_ref[...], jnp.zeros((BLK, 5), jnp.float32)], axis=1)
    r_ref[...] = r


def _tc_mid_body(nchunk, with_sum, *refs):
    m_refs = refs[:nchunk]
    r_ref, w_ref, b_ref = refs[nchunk:nchunk + 3]
    out_refs = refs[nchunk + 3:]
    m = jnp.concatenate([mr[...] for mr in m_refs], axis=1) * r_ref[...]
    h = jax.nn.relu(jnp.dot(m, w_ref[...],
                            preferred_element_type=jnp.float32) + b_ref[...])
    wout = h.shape[1]
    nout = len(out_refs) - (1 if with_sum else 0)
    cw = wout // nout
    for k in range(nout):
        out_refs[k][...] = h[:, k * cw:(k + 1) * cw]
    if with_sum:
        i = pl.program_id(0)
        part = jnp.concatenate(
            [jnp.sum(h, axis=0, keepdims=True),
             jnp.zeros((7, wout), jnp.float32)], axis=0)

        @pl.when(i == 0)
        def _():
            out_refs[-1][...] = jnp.zeros((8, wout), jnp.float32)

        out_refs[-1][...] += part


def _tc5_body(m_ref, r_ref, wc_ref, bc_ref, cs_ref, wg1_ref, bg1_ref,
              wg2_ref, bg2_ref, nl_ref, gl_ref):
    m = m_ref[...] * r_ref[...]
    nl_ref[...] = jnp.dot(m, wc_ref[...],
                          preferred_element_type=jnp.float32) + bc_ref[...]
    i = pl.program_id(0)

    @pl.when(i == 0)
    def _():
        x = cs_ref[0:1, :] * (1.0 / NN)
        g1 = jax.nn.relu(jnp.dot(x, wg1_ref[...],
                                 preferred_element_type=jnp.float32) + bg1_ref[...])
        g2 = jnp.dot(g1, wg2_ref[...],
                     preferred_element_type=jnp.float32) + bg2_ref[...]
        gl_ref[...] = jax.nn.sigmoid(g2)


def kernel(edge_attr, node_info, edge_index, W0, b0, W1, b1, W2, b2, W3, b3,
           Wc, bc, Wg1, bg1, Wg2, bg2):
    src = edge_index[0]
    dst = edge_index[1]
    pad = E_PAD - EE
    src_p = jnp.concatenate([src, jnp.zeros((pad,), jnp.int32)]).reshape(R_PAD, 128)
    dst_p = jnp.concatenate([dst, jnp.full((pad,), NN, jnp.int32)]).reshape(R_PAD, 128)
    ea16 = jnp.concatenate(
        [edge_attr, jnp.ones((EE, 1), jnp.float32),
         jnp.zeros((EE, 10), jnp.float32)], axis=1)
    ea16 = jnp.concatenate([ea16, jnp.zeros((pad, 16), jnp.float32)], axis=0)

    dstl, m0deg = _sc_prep_pass0(dst_p, ea16)

    c0, c1, c2, recip = pl.pallas_call(
        _tc1_body,
        grid=(GRID,),
        in_specs=[_row_spec(16), _row_spec(11), _full_spec((5, 64)),
                  _full_spec((64,))],
        out_specs=[_row_spec(32), _row_spec(32), _row_spec(16), _row_spec(1)],
        out_shape=[jax.ShapeDtypeStruct((NN, 32), jnp.float32),
                   jax.ShapeDtypeStruct((NN, 32), jnp.float32),
                   jax.ShapeDtypeStruct((NN, 16), jnp.float32),
                   jax.ShapeDtypeStruct((NN, 1), jnp.float32)],
    )(m0deg, node_info, W0, b0)

    m1c0 = _sc_pass32(src_p, dstl, c0)
    m1c1 = _sc_pass32(src_p, dstl, c1)
    m1c2 = _sc_pass16(src_p, dstl, c2)

    W1p = jnp.concatenate([W1, jnp.zeros((5, 64), jnp.float32)], axis=0)
    d0, d1 = pl.pallas_call(
        functools.partial(_tc_mid_body, 3, False),
        grid=(GRID,),
        in_specs=[_row_spec(32), _row_spec(32), _row_spec(16), _row_spec(1),
                  _full_spec((80, 64)), _full_spec((64,))],
        out_specs=[_row_spec(32), _row_spec(32)],
        out_shape=[jax.ShapeDtypeStruct((NN, 32), jnp.float32),
                   jax.ShapeDtypeStruct((NN, 32), jnp.float32)],
    )(m1c0, m1c1, m1c2, recip, W1p, b1)

    m2c0 = _sc_pass32(src_p, dstl, d0)
    m2c1 = _sc_pass32(src_p, dstl, d1)

    (e0,) = pl.pallas_call(
        functools.partial(_tc_mid_body, 2, False),
        grid=(GRID,),
        in_specs=[_row_spec(32), _row_spec(32), _row_spec(1),
                  _full_spec((64, 32)), _full_spec((32,))],
        out_specs=[_row_spec(32)],
        out_shape=[jax.ShapeDtypeStruct((NN, 32), jnp.float32)],
    )(m2c0, m2c1, recip, W2, b2)

    m3 = _sc_pass32(src_p, dstl, e0)

    f0, colsum = pl.pallas_call(
        functools.partial(_tc_mid_body, 1, True),
        grid=(GRID,),
        in_specs=[_row_spec(32), _row_spec(1),
                  _full_spec((32, 16)), _full_spec((16,))],
        out_specs=[_row_spec(16), _full_spec((8, 16))],
        out_shape=[jax.ShapeDtypeStruct((NN, 16), jnp.float32),
                   jax.ShapeDtypeStruct((8, 16), jnp.float32)],
    )(m3, recip, W3, b3)

    m4 = _sc_pass16(src_p, dstl, f0)

    node_label, graph_label = pl.pallas_call(
        _tc5_body,
        grid=(GRID,),
        in_specs=[_row_spec(16), _row_spec(1), _full_spec((16, 2)),
                  _full_spec((2,)), _full_spec((8, 16)), _full_spec((16, 8)),
                  _full_spec((8,)), _full_spec((8, 1)), _full_spec((1,))],
        out_specs=[_row_spec(2), _full_spec((1, 1))],
        out_shape=[jax.ShapeDtypeStruct((NN, 2), jnp.float32),
                   jax.ShapeDtypeStruct((1, 1), jnp.float32)],
    )(m4, recip, Wc, bc, colsum, Wg1, bg1, Wg2, bg2)

    return (graph_label, node_label)


# speedup vs baseline: 3.3185x; 3.3185x over previous
import functools

import jax
import jax.numpy as jnp
from jax import lax
from jax.experimental import pallas as pl
from jax.experimental.pallas import tpu as pltpu
from jax.experimental.pallas import tpu_sc as plsc

NN = 100000
EE = 1600000
NC, NS = 2, 16
NHALF = NN // 2
TRASH = NHALF
ACC_ROWS = NHALF + 8
RPT = 784
R_PAD = RPT * NS
E_PAD = R_PAD * 128
RPN = NHALF // NS
CB = 49
NG = RPT // CB

_SC_MESH = plsc.VectorSubcoreMesh(core_axis_name="c", subcore_axis_name="s")
_SC_PARAMS = pltpu.CompilerParams(use_tc_tiling_on_sc=False)


@functools.partial(
    pl.kernel,
    out_type=pltpu.MemorySpace.HBM((NC, R_PAD, 128), jnp.int32),
    mesh=_SC_MESH,
    scratch_types=[pltpu.VMEM((CB, 128), jnp.int32)],
    compiler_params=_SC_PARAMS,
)
def _sc_prep(dst_hbm, dstl_hbm, idx):
    c = lax.axis_index("c")
    s = lax.axis_index("s")
    base = c * NHALF
    for g in range(NG):
        r0 = s * RPT + g * CB
        pltpu.sync_copy(dst_hbm.at[pl.ds(r0, CB)], idx)

        def clamp_row(j, _):
            for k in range(8):
                d16 = idx[j, pl.ds(k * 16, 16)]
                v = d16 - base
                ok = (v >= 0) & (v < NHALF)
                idx[j, pl.ds(k * 16, 16)] = jnp.where(ok, v, TRASH)
            return 0

        lax.fori_loop(0, CB, clamp_row, 0)
        pltpu.sync_copy(idx, dstl_hbm.at[c, pl.ds(r0, CB)])


def _make_sc_pass(width, x_rows):
    @functools.partial(
        pl.kernel,
        out_type=pltpu.MemorySpace.HBM((NN, width), jnp.float32),
        mesh=_SC_MESH,
        scratch_types=[
            pltpu.VMEM_SHARED((ACC_ROWS, width), jnp.float32),
            pltpu.VMEM((CB, 128), jnp.int32),
            pltpu.VMEM((CB, 128), jnp.int32),
            pltpu.VMEM((128, width), jnp.float32),
            pltpu.VMEM((128, width), jnp.float32),
            pltpu.SemaphoreType.DMA,
        ],
        compiler_params=_SC_PARAMS,
    )
    def sc_pass(src_hbm, dstl_hbm, x_hbm, m_hbm, acc, srcv, dstv, rows, zbuf, sem):
        c = lax.axis_index("c")
        s = lax.axis_index("s")
        for r in range(128):
            for k in range(width // 16):
                zbuf[r, pl.ds(k * 16, 16)] = jnp.zeros((16,), jnp.float32)
        nfull, rem = RPN // 128, RPN % 128
        for i in range(nfull):
            pltpu.sync_copy(zbuf, acc.at[pl.ds(s * RPN + i * 128, 128)])
        if rem:
            pltpu.sync_copy(zbuf.at[pl.ds(0, rem)],
                            acc.at[pl.ds(s * RPN + nfull * 128, rem)])

        @pl.when(s == 0)
        def _():
            pltpu.sync_copy(zbuf.at[pl.ds(0, 8)], acc.at[pl.ds(NHALF, 8)])

        plsc.subcore_barrier()
        for g in range(NG):
            r0 = s * RPT + g * CB
            pltpu.sync_copy(src_hbm.at[pl.ds(r0, CB)], srcv)
            pltpu.sync_copy(dstl_hbm.at[c, pl.ds(r0, CB)], dstv)

            def body(j, _):
                pltpu.async_copy(x_hbm.at[srcv.at[j]], rows, sem).wait()
                pltpu.sync_copy(rows, acc.at[dstv.at[j]], add=True)
                return 0

            lax.fori_loop(0, CB, body, 0)
        plsc.subcore_barrier()
        pltpu.sync_copy(acc.at[pl.ds(s * RPN, RPN)],
                        m_hbm.at[pl.ds(c * NHALF + s * RPN, RPN)])

    def wrapped(src, dstl, x):
        assert x.shape == (x_rows, width)
        return sc_pass(src, dstl, x)

    return wrapped


_sc_passE16 = _make_sc_pass(16, E_PAD)
_sc_pass32 = _make_sc_pass(32, NN)
_sc_pass16 = _make_sc_pass(16, NN)

BLK = 1000
GRID = NN // BLK


def _row_spec(w):
    return pl.BlockSpec((BLK, w), lambda i: (i, 0))


def _full_spec(shape):
    nd = len(shape)
    return pl.BlockSpec(shape, lambda i: (0,) * nd)


def _tc1_body(md_ref, ni_ref, w0_ref, b0_ref, c0_ref, c1_ref, c2_ref, r_ref):
    md = md_ref[...]
    r = 1.0 / jnp.maximum(md[:, 5:6], 1.0)
    m0 = md[:, 0:5] * r
    h = jax.nn.relu(jnp.dot(m0, w0_ref[...],
                            preferred_element_type=jnp.float32) + b0_ref[...])
    c0_ref[...] = h[:, :32]
    c1_ref[...] = h[:, 32:]
    c2_ref[...] = jnp.concatenate(
        [ni_ref[...], jnp.zeros((BLK, 5), jnp.float32)], axis=1)
    r_ref[...] = r


def _tc_mid_body(nchunk, with_sum, *refs):
    m_refs = refs[:nchunk]
    r_ref, w_ref, b_ref = refs[nchunk:nchunk + 3]
    out_refs = refs[nchunk + 3:]
    m = jnp.concatenate([mr[...] for mr in m_refs], axis=1) * r_ref[...]
    h = jax.nn.relu(jnp.dot(m, w_ref[...],
                            preferred_element_type=jnp.float32) + b_ref[...])
    wout = h.shape[1]
    nout = len(out_refs) - (1 if with_sum else 0)
    cw = wout // nout
    for k in range(nout):
        out_refs[k][...] = h[:, k * cw:(k + 1) * cw]
    if with_sum:
        i = pl.program_id(0)
        part = jnp.concatenate(
            [jnp.sum(h, axis=0, keepdims=True),
             jnp.zeros((7, wout), jnp.float32)], axis=0)

        @pl.when(i == 0)
        def _():
            out_refs[-1][...] = jnp.zeros((8, wout), jnp.float32)

        out_refs[-1][...] += part


def _tc5_body(m_ref, r_ref, wc_ref, bc_ref, cs_ref, wg1_ref, bg1_ref,
              wg2_ref, bg2_ref, nl_ref, gl_ref):
    m = m_ref[...] * r_ref[...]
    nl_ref[...] = jnp.dot(m, wc_ref[...],
                          preferred_element_type=jnp.float32) + bc_ref[...]
    i = pl.program_id(0)

    @pl.when(i == 0)
    def _():
        x = cs_ref[0:1, :] * (1.0 / NN)
        g1 = jax.nn.relu(jnp.dot(x, wg1_ref[...],
                                 preferred_element_type=jnp.float32) + bg1_ref[...])
        g2 = jnp.dot(g1, wg2_ref[...],
                     preferred_element_type=jnp.float32) + bg2_ref[...]
        gl_ref[...] = jax.nn.sigmoid(g2)


def kernel(edge_attr, node_info, edge_index, W0, b0, W1, b1, W2, b2, W3, b3,
           Wc, bc, Wg1, bg1, Wg2, bg2):
    src = edge_index[0]
    dst = edge_index[1]
    pad = E_PAD - EE
    src_p = jnp.concatenate([src, jnp.zeros((pad,), jnp.int32)]).reshape(R_PAD, 128)
    dst_p = jnp.concatenate([dst, jnp.full((pad,), NN, jnp.int32)]).reshape(R_PAD, 128)
    eidx = jnp.arange(E_PAD, dtype=jnp.int32).reshape(R_PAD, 128)
    ea16 = jnp.concatenate(
        [edge_attr, jnp.ones((EE, 1), jnp.float32),
         jnp.zeros((EE, 10), jnp.float32)], axis=1)
    ea16 = jnp.concatenate([ea16, jnp.zeros((pad, 16), jnp.float32)], axis=0)

    dstl = _sc_prep(dst_p)
    m0deg = _sc_passE16(eidx, dstl, ea16)

    c0, c1, c2, recip = pl.pallas_call(
        _tc1_body,
        grid=(GRID,),
        in_specs=[_row_spec(16), _row_spec(11), _full_spec((5, 64)),
                  _full_spec((64,))],
        out_specs=[_row_spec(32), _row_spec(32), _row_spec(16), _row_spec(1)],
        out_shape=[jax.ShapeDtypeStruct((NN, 32), jnp.float32),
                   jax.ShapeDtypeStruct((NN, 32), jnp.float32),
                   jax.ShapeDtypeStruct((NN, 16), jnp.float32),
                   jax.ShapeDtypeStruct((NN, 1), jnp.float32)],
    )(m0deg, node_info, W0, b0)

    m1c0 = _sc_pass32(src_p, dstl, c0)
    m1c1 = _sc_pass32(src_p, dstl, c1)
    m1c2 = _sc_pass16(src_p, dstl, c2)

    W1p = jnp.concatenate([W1, jnp.zeros((5, 64), jnp.float32)], axis=0)
    d0, d1 = pl.pallas_call(
        functools.partial(_tc_mid_body, 3, False),
        grid=(GRID,),
        in_specs=[_row_spec(32), _row_spec(32), _row_spec(16), _row_spec(1),
                  _full_spec((80, 64)), _full_spec((64,))],
        out_specs=[_row_spec(32), _row_spec(32)],
        out_shape=[jax.ShapeDtypeStruct((NN, 32), jnp.float32),
                   jax.ShapeDtypeStruct((NN, 32), jnp.float32)],
    )(m1c0, m1c1, m1c2, recip, W1p, b1)

    m2c0 = _sc_pass32(src_p, dstl, d0)
    m2c1 = _sc_pass32(src_p, dstl, d1)

    (e0,) = pl.pallas_call(
        functools.partial(_tc_mid_body, 2, False),
        grid=(GRID,),
        in_specs=[_row_spec(32), _row_spec(32), _row_spec(1),
                  _full_spec((64, 32)), _full_spec((32,))],
        out_specs=[_row_spec(32)],
        out_shape=[jax.ShapeDtypeStruct((NN, 32), jnp.float32)],
    )(m2c0, m2c1, recip, W2, b2)

    m3 = _sc_pass32(src_p, dstl, e0)

    f0, colsum = pl.pallas_call(
        functools.partial(_tc_mid_body, 1, True),
        grid=(GRID,),
        in_specs=[_row_spec(32), _row_spec(1),
                  _full_spec((32, 16)), _full_spec((16,))],
        out_specs=[_row_spec(16), _full_spec((8, 16))],
        out_shape=[jax.ShapeDtypeStruct((NN, 16), jnp.float32),
                   jax.ShapeDtypeStruct((8, 16), jnp.float32)],
    )(m3, recip, W3, b3)

    m4 = _sc_pass16(src_p, dstl, f0)

    node_label, graph_label = pl.pallas_call(
        _tc5_body,
        grid=(GRID,),
        in_specs=[_row_spec(16), _row_spec(1), _full_spec((16, 2)),
                  _full_spec((2,)), _full_spec((8, 16)), _full_spec((16, 8)),
                  _full_spec((8,)), _full_spec((8, 1)), _full_spec((1,))],
        out_specs=[_row_spec(2), _full_spec((1, 1))],
        out_shape=[jax.ShapeDtypeStruct((NN, 2), jnp.float32),
                   jax.ShapeDtypeStruct((1, 1), jnp.float32)],
    )(m4, recip, Wc, bc, colsum, Wg1, bg1, Wg2, bg2)

    return (graph_label, node_label)
